# baseline (speedup 1.0000x reference)
import functools

import jax
import jax.numpy as jnp
import numpy as np
from jax import lax
from jax.experimental import pallas as pl
from jax.experimental.pallas import tpu as pltpu
from jax.experimental.pallas import tpu_sc as plsc

_ANCHORS = [1.3221, 1.73145, 3.19275, 4.00944, 5.05587, 8.09892,
            9.47112, 4.84053, 11.2364, 10.0071]
_A = 5
_OBJECT_SCALE = 5.0
_SIL_THRESH = 0.6
_L = 16

_F32 = jnp.float32
_I32 = jnp.int32


def _bcast_lane(v, i):
    idx = jnp.full((_L,), i, _I32)
    dnums = lax.GatherDimensionNumbers(
        offset_dims=(), collapsed_slice_dims=(0,), start_index_map=(0,))
    return lax.gather(v, idx[:, None], dnums, slice_sizes=(1,),
                      mode=lax.GatherScatterMode.PROMISE_IN_BOUNDS)


def _sig(x):
    return 1.0 / (1.0 + jnp.exp(-x))


def _vlog(x):
    bits = lax.bitcast_convert_type(x, _I32)
    e = (bits >> 23) - 127
    mbits = (bits & _I32(0x007FFFFF)) | _I32(0x3F800000)
    m = lax.bitcast_convert_type(mbits, _F32)
    big = m > 1.41421356237
    m = jnp.where(big, m * 0.5, m)
    e = e + jnp.where(big, 1, 0)
    t = m - 1.0
    z = t * t
    p = jnp.full((_L,), 7.0376836292e-2, _F32)
    for c in (-1.1514610310e-1, 1.1676998740e-1, -1.2420140846e-1,
              1.4249322787e-1, -1.6668057665e-1, 2.0000714765e-1,
              -2.4999993993e-1, 3.3333331174e-1):
        p = p * t + _F32(c)
    y = t * z * p - 0.5 * z
    return t + y + e.astype(_F32) * _F32(0.6931471805599453)


def _build_sc_call(B, H, W, U):
    HW = H * W
    ROW = _A * 5 * HW
    ROWP = ((ROW + 15) // 16) * 16
    nfull = HW // _L
    NCH = nfull + (1 if HW % _L else 0)
    GRID = NCH * _L
    try:
        info = plsc.get_sparse_core_info()
        NC, NS = info.num_cores, info.num_subcores
    except Exception:
        NC, NS = 2, 16
    NW = NC * NS
    BPW = B // NW
    UW = U * BPW
    UWIN = ((UW + 10 + 15) // 16) * 16
    UWIN = UW + 10
    NUF = UW // _L
    T0 = 0
    P0 = 4 * B
    G0 = P0 + U * B
    STOT = G0 + U * B

    mesh = plsc.VectorSubcoreMesh(core_axis_name="c", subcore_axis_name="s")

    @functools.partial(
        pl.kernel, mesh=mesh,
        out_type=jax.ShapeDtypeStruct((NW, _L), _F32),
        scratch_types=[
            pltpu.VMEM((ROWP,), _F32),
            pltpu.VMEM((_L,), _F32),
            pltpu.VMEM((UWIN,), _F32),
            pltpu.VMEM((UWIN,), _F32),
            pltpu.VMEM((_L,), _F32),
            pltpu.VMEM((4 * GRID,), _F32),
        ],
    )
    def sc_loss(pred_hbm, sm_hbm, gr_hbm, out_hbm,
                pred_v, targ_v, pu_v, gu_v, out_v, gr_v):
        wid = lax.axis_index("s") * NC + lax.axis_index("c")
        zero = jnp.zeros((_L,), _F32)
        acc = zero

        lanev = lax.iota(_I32, _L)
        pltpu.sync_copy(gr_hbm, gr_v)

        for k in range(BPW):
            b = wid * BPW + k
            pltpu.sync_copy(pred_hbm.at[b], pred_v)
            tb = b * 4
            tsh = lax.rem(tb, 8)
            pltpu.sync_copy(sm_hbm.at[pl.ds(pl.multiple_of(tb - tsh, 8), _L)], targ_v)
            tv = targ_v[...]
            gxv = _bcast_lane(tv, tsh) * _F32(W)
            gyv = _bcast_lane(tv, tsh + 1) * _F32(H)
            gwv = _bcast_lane(tv, tsh + 2) * _F32(W)
            ghv = _bcast_lane(tv, tsh + 3) * _F32(H)
            gxl = gxv - gwv * 0.5
            gxr = gxv + gwv * 0.5
            gyl = gyv - ghv * 0.5
            gyr = gyv + ghv * 0.5
            garea = gwv * ghv

            bestv = jnp.zeros((_L,), _I32)
            biou = None
            for a in range(_A):
                awa = _ANCHORS[2 * a]
                aha = _ANCHORS[2 * a + 1]
                uw = jnp.maximum(gwv, _F32(awa))
                uh = jnp.maximum(ghv, _F32(aha))
                cw = (gwv + _F32(awa)) - uw
                ch = (ghv + _F32(aha)) - uh
                carea = jnp.maximum(cw, 0.0) * jnp.maximum(ch, 0.0)
                uarea = (_F32(awa * aha) + garea) - carea
                au = carea / uarea
                if biou is None:
                    biou = au
                else:
                    upd = au > biou
                    bestv = jnp.where(upd, a, bestv)
                    biou = jnp.where(upd, au, biou)
            awbv = zero
            ahbv = zero
            for a in range(_A):
                hit = bestv == a
                awbv = awbv + jnp.where(hit, _F32(_ANCHORS[2 * a]), 0.0)
                ahbv = ahbv + jnp.where(hit, _F32(_ANCHORS[2 * a + 1]), 0.0)
            lwv = _vlog(gwv / awbv)
            lhv = _vlog(ghv / ahbv)
            giv = gxv.astype(_I32)
            gjv = gyv.astype(_I32)
            dxv = gxv - giv.astype(_F32)
            dyv = gyv - gjv.astype(_F32)
            pselv = (gjv * W + giv).astype(_F32)

            def chunka(t, acc, gxl=gxl, gxr=gxr, gyl=gyl, gyr=gyr,
                       garea=garea, gwv=gwv, ghv=ghv, dxv=dxv, dyv=dyv,
                       lwv=lwv, lhv=lhv, pselv=pselv, bestv=bestv):
                a = lax.div(t, NCH * _L)
                go = t - a * (NCH * _L)
                awa = jnp.full((_L,), _ANCHORS[2 * (_A - 1)], _F32)
                aha = jnp.full((_L,), _ANCHORS[2 * (_A - 1) + 1], _F32)
                for aa in range(_A - 1):
                    awa = jnp.where(a == aa, _F32(_ANCHORS[2 * aa]), awa)
                    aha = jnp.where(a == aa, _F32(_ANCHORS[2 * aa + 1]), aha)
                bm = jnp.where(bestv == a, _F32(1.0), _F32(0.0))
                off = jnp.minimum(go, HW - _L) + a * (5 * HW)
                xr = pred_v[pl.ds(off, _L)]
                yr = pred_v[pl.ds(off + HW, _L)]
                twv = pred_v[pl.ds(off + 2 * HW, _L)]
                thv = pred_v[pl.ds(off + 3 * HW, _L)]
                cr = pred_v[pl.ds(off + 4 * HW, _L)]
                wg = gr_v[pl.ds(go, _L)]
                hg = gr_v[pl.ds(go + GRID, _L)]
                psc = gr_v[pl.ds(go + 2 * GRID, _L)]
                vm = gr_v[pl.ds(go + 3 * GRID, _L)]
                ax = 1.0 + jnp.exp(-xr)
                ay = 1.0 + jnp.exp(-yr)
                acf = 1.0 + jnp.exp(-cr)
                axy = ax * ay
                bwv = jnp.exp(twv) * awa
                bhv = jnp.exp(thv) * aha
                rx = 1.0 / axy
                sx = rx * ay
                sy = rx * ax
                bxv = sx + wg
                byv = sy + hg
                mx = jnp.minimum(bxv - bwv * 0.5, gxl)
                nx = jnp.maximum(bxv + bwv * 0.5, gxr)
                my = jnp.minimum(byv - bhv * 0.5, gyl)
                ny = jnp.maximum(byv + bhv * 0.5, gyr)
                cw = (bwv + gwv) - (nx - mx)
                ch = (bhv + ghv) - (ny - my)
                carea = jnp.maximum(cw, 0.0) * jnp.maximum(ch, 0.0)
                uarea = (bwv * bhv + garea) - carea
                den = acf * uarea
                rcu = 1.0 / den
                cf = rcu * uarea
                iou = (rcu * acf) * carea
                m01 = jnp.where(iou > _SIL_THRESH, _F32(0.0), vm)
                sxc = sx - 0.5
                syc = sy - 0.5
                base = sxc * sxc + syc * syc + twv * twv + thv * thv
                cfm = cf * cf * m01
                sel = jnp.where(psc == pselv, bm, _F32(0.0))
                ex = sx - dxv
                ey = sy - dyv
                ew = twv - lwv
                eh = thv - lhv
                ec = cf - iou
                quad = (ex * ex + ey * ey + ew * ew + eh * eh
                        + _OBJECT_SCALE * (ec * ec))
                corr = quad - base - cfm
                return acc + base * vm + cfm + sel * corr

            acc = plsc.parallel_loop(0, _A * NCH * _L, _L, unroll=2,
                                     carry=acc)(chunka)

        ub = wid * UW
        pb = P0 + ub
        psh = lax.rem(pb, 8)
        pltpu.sync_copy(sm_hbm.at[pl.ds(pl.multiple_of(pb - psh, 8), UWIN)], pu_v)
        gb = G0 + ub
        gal = pl.multiple_of(jnp.minimum(gb - lax.rem(gb, 8), STOT - UWIN), 8)
        gsh = gb - gal
        pltpu.sync_copy(sm_hbm.at[pl.ds(gal, UWIN)], gu_v)
        for c in range(NUF):
            dv = (gu_v[pl.ds(gsh + c * _L, _L)]
                  - pu_v[pl.ds(psh + c * _L, _L)])
            acc = acc + dv * dv
        tail = UW - NUF * _L
        if tail:
            um = jnp.where(lanev < _L - tail, _F32(0.0), _F32(1.0))
            dv = (gu_v[pl.ds(gsh + UW - _L, _L)]
                  - pu_v[pl.ds(psh + UW - _L, _L)]) * um
            acc = acc + dv * dv

        out_v[...] = acc * 0.5
        pltpu.sync_copy(out_v, out_hbm.at[wid])

    return sc_loss


def kernel(pred, pred_uvd, target, uvd_gt, train_out):
    B, H, W = pred.shape[0], pred.shape[2], pred.shape[3]
    U = pred_uvd.shape[1]
    sc_loss = _build_sc_call(B, H, W, U)
    ROW = _A * 5 * H * W
    ROWP = ((ROW + 15) // 16) * 16
    HW = H * W
    NCH = HW // _L + (1 if HW % _L else 0)
    wg, hg, ps, vm = [], [], [], []
    for j in range(NCH):
        off = min(j * _L, HW - _L)
        ndup = j * _L - off
        for i in range(_L):
            p = off + i
            wg.append(float(p % W))
            hg.append(float(p // W))
            ps.append(-1.0 if i < ndup else float(p))
            vm.append(0.0 if i < ndup else 1.0)
    gridc = jnp.asarray(np.asarray(wg + hg + ps + vm, np.float32))
    predf = jnp.pad(pred.reshape(B, ROW), ((0, 0), (0, ROWP - ROW)))
    smalls = jnp.concatenate(
        [target.reshape(-1), pred_uvd.reshape(-1), uvd_gt.reshape(-1)])
    partials = sc_loss(predf, smalls, gridc)
    return jnp.sum(partials)


# speedup vs baseline: 1.0644x; 1.0644x over previous
import functools

import jax
import jax.numpy as jnp
from jax import lax
from jax.experimental import pallas as pl
from jax.experimental.pallas import tpu as pltpu
from jax.experimental.pallas import tpu_sc as plsc

_ANCHORS = [1.3221, 1.73145, 3.19275, 4.00944, 5.05587, 8.09892,
            9.47112, 4.84053, 11.2364, 10.0071]
_A = 5
_OBJECT_SCALE = 5.0
_SIL_THRESH = 0.6
_L = 16

_F32 = jnp.float32
_I32 = jnp.int32


def _bcast_lane(v, i):
    idx = jnp.full((_L,), i, _I32)
    dnums = lax.GatherDimensionNumbers(
        offset_dims=(), collapsed_slice_dims=(0,), start_index_map=(0,))
    return lax.gather(v, idx[:, None], dnums, slice_sizes=(1,),
                      mode=lax.GatherScatterMode.PROMISE_IN_BOUNDS)


def _sig(x):
    return 1.0 / (1.0 + jnp.exp(-x))


def _vlog(x):
    bits = lax.bitcast_convert_type(x, _I32)
    e = (bits >> 23) - 127
    mbits = (bits & _I32(0x007FFFFF)) | _I32(0x3F800000)
    m = lax.bitcast_convert_type(mbits, _F32)
    big = m > 1.41421356237
    m = jnp.where(big, m * 0.5, m)
    e = e + jnp.where(big, 1, 0)
    t = m - 1.0
    z = t * t
    p = jnp.full((_L,), 7.0376836292e-2, _F32)
    for c in (-1.1514610310e-1, 1.1676998740e-1, -1.2420140846e-1,
              1.4249322787e-1, -1.6668057665e-1, 2.0000714765e-1,
              -2.4999993993e-1, 3.3333331174e-1):
        p = p * t + _F32(c)
    y = t * z * p - 0.5 * z
    return t + y + e.astype(_F32) * _F32(0.6931471805599453)


def _build_sc_call(B, H, W, U):
    HW = H * W
    ROW = _A * 5 * HW
    ROWP = ((ROW + 15) // 16) * 16
    nfull = HW // _L
    NCH = nfull + (1 if HW % _L else 0)
    GRID = NCH * _L
    try:
        info = plsc.get_sparse_core_info()
        NC, NS = info.num_cores, info.num_subcores
    except Exception:
        NC, NS = 2, 16
    NW = NC * NS
    BPW = B // NW
    UW = U * BPW
    UWIN = ((UW + 10 + 15) // 16) * 16
    UWIN = UW + 10
    NUF = UW // _L
    T0 = 0
    P0 = 4 * B
    G0 = P0 + U * B
    STOT = G0 + U * B

    mesh = plsc.VectorSubcoreMesh(core_axis_name="c", subcore_axis_name="s")

    @functools.partial(
        pl.kernel, mesh=mesh,
        out_type=jax.ShapeDtypeStruct((NW, _L), _F32),
        scratch_types=[
            pltpu.VMEM((ROWP,), _F32),
            pltpu.VMEM((_L,), _F32),
            pltpu.VMEM((UWIN,), _F32),
            pltpu.VMEM((UWIN,), _F32),
            pltpu.VMEM((_L,), _F32),
            pltpu.VMEM((GRID,), _F32),
            pltpu.VMEM((GRID,), _F32),
            pltpu.VMEM((GRID,), _I32),
            pltpu.VMEM((GRID,), _F32),
        ],
    )
    def sc_loss(pred_hbm, sm_hbm, out_hbm,
                pred_v, targ_v, pu_v, gu_v, out_v,
                wg_v, hg_v, psc_v, vm_v):
        wid = lax.axis_index("s") * NC + lax.axis_index("c")
        zero = jnp.zeros((_L,), _F32)
        acc = zero

        lanev = lax.iota(_I32, _L)
        ones = jnp.full((_L,), 1.0, _F32)
        for j in range(NCH):
            off = min(j * _L, HW - _L)
            pos = lanev + off
            ndup = j * _L - off
            if j * _L + _L > HW:
                psc_v[pl.ds(j * _L, _L)] = jnp.where(lanev < ndup, -1, pos)
                vm_v[pl.ds(j * _L, _L)] = jnp.where(lanev < ndup, _F32(0.0),
                                                    _F32(1.0))
            else:
                psc_v[pl.ds(j * _L, _L)] = pos
                vm_v[pl.ds(j * _L, _L)] = ones
            wg_v[pl.ds(j * _L, _L)] = lax.rem(pos, W).astype(_F32)
            hg_v[pl.ds(j * _L, _L)] = lax.div(pos, W).astype(_F32)

        for k in range(BPW):
            b = wid * BPW + k
            pltpu.sync_copy(pred_hbm.at[b], pred_v)
            tb = b * 4
            tsh = lax.rem(tb, 8)
            pltpu.sync_copy(sm_hbm.at[pl.ds(pl.multiple_of(tb - tsh, 8), _L)], targ_v)
            tv = targ_v[...]
            gxv = _bcast_lane(tv, tsh) * _F32(W)
            gyv = _bcast_lane(tv, tsh + 1) * _F32(H)
            gwv = _bcast_lane(tv, tsh + 2) * _F32(W)
            ghv = _bcast_lane(tv, tsh + 3) * _F32(H)
            gxl = gxv - gwv * 0.5
            gxr = gxv + gwv * 0.5
            gyl = gyv - ghv * 0.5
            gyr = gyv + ghv * 0.5
            garea = gwv * ghv

            bestv = jnp.zeros((_L,), _I32)
            biou = None
            for a in range(_A):
                awa = _ANCHORS[2 * a]
                aha = _ANCHORS[2 * a + 1]
                uw = jnp.maximum(gwv, _F32(awa))
                uh = jnp.maximum(ghv, _F32(aha))
                cw = (gwv + _F32(awa)) - uw
                ch = (ghv + _F32(aha)) - uh
                carea = jnp.maximum(cw, 0.0) * jnp.maximum(ch, 0.0)
                uarea = (_F32(awa * aha) + garea) - carea
                au = carea / uarea
                if biou is None:
                    biou = au
                else:
                    upd = au > biou
                    bestv = jnp.where(upd, a, bestv)
                    biou = jnp.where(upd, au, biou)
            awbv = zero
            ahbv = zero
            for a in range(_A):
                hit = bestv == a
                awbv = awbv + jnp.where(hit, _F32(_ANCHORS[2 * a]), 0.0)
                ahbv = ahbv + jnp.where(hit, _F32(_ANCHORS[2 * a + 1]), 0.0)
            lwv = _vlog(gwv / awbv)
            lhv = _vlog(ghv / ahbv)
            giv = gxv.astype(_I32)
            gjv = gyv.astype(_I32)
            dxv = gxv - giv.astype(_F32)
            dyv = gyv - gjv.astype(_F32)
            pselv = gjv * W + giv

            def chunka(t, acc, gxl=gxl, gxr=gxr, gyl=gyl, gyr=gyr,
                       garea=garea, gwv=gwv, ghv=ghv, dxv=dxv, dyv=dyv,
                       lwv=lwv, lhv=lhv, pselv=pselv, bestv=bestv):
                a = lax.div(t, NCH * _L)
                go = t - a * (NCH * _L)
                awa = jnp.full((_L,), _ANCHORS[2 * (_A - 1)], _F32)
                aha = jnp.full((_L,), _ANCHORS[2 * (_A - 1) + 1], _F32)
                for aa in range(_A - 1):
                    awa = jnp.where(a == aa, _F32(_ANCHORS[2 * aa]), awa)
                    aha = jnp.where(a == aa, _F32(_ANCHORS[2 * aa + 1]), aha)
                bm = jnp.where(bestv == a, _F32(1.0), _F32(0.0))
                off = jnp.minimum(go, HW - _L) + a * (5 * HW)
                xr = pred_v[pl.ds(off, _L)]
                yr = pred_v[pl.ds(off + HW, _L)]
                twv = pred_v[pl.ds(off + 2 * HW, _L)]
                thv = pred_v[pl.ds(off + 3 * HW, _L)]
                cr = pred_v[pl.ds(off + 4 * HW, _L)]
                wg = wg_v[pl.ds(go, _L)]
                hg = hg_v[pl.ds(go, _L)]
                psc = psc_v[pl.ds(go, _L)]
                vm = vm_v[pl.ds(go, _L)]
                ax = 1.0 + jnp.exp(-xr)
                ay = 1.0 + jnp.exp(-yr)
                acf = 1.0 + jnp.exp(-cr)
                axy = ax * ay
                bwv = jnp.exp(twv) * awa
                bhv = jnp.exp(thv) * aha
                rx = 1.0 / axy
                sx = rx * ay
                sy = rx * ax
                bxv = sx + wg
                byv = sy + hg
                mx = jnp.minimum(bxv - bwv * 0.5, gxl)
                nx = jnp.maximum(bxv + bwv * 0.5, gxr)
                my = jnp.minimum(byv - bhv * 0.5, gyl)
                ny = jnp.maximum(byv + bhv * 0.5, gyr)
                cw = (bwv + gwv) - (nx - mx)
                ch = (bhv + ghv) - (ny - my)
                carea = jnp.maximum(cw, 0.0) * jnp.maximum(ch, 0.0)
                uarea = (bwv * bhv + garea) - carea
                den = acf * uarea
                rcu = 1.0 / den
                cf = rcu * uarea
                iou = (rcu * acf) * carea
                m01 = jnp.where(iou > _SIL_THRESH, _F32(0.0), vm)
                sxc = sx - 0.5
                syc = sy - 0.5
                base = sxc * sxc + syc * syc + twv * twv + thv * thv
                cfm = cf * cf * m01
                sel = jnp.where(psc == pselv, bm, _F32(0.0))
                ex = sx - dxv
                ey = sy - dyv
                ew = twv - lwv
                eh = thv - lhv
                ec = cf - iou
                quad = (ex * ex + ey * ey + ew * ew + eh * eh
                        + _OBJECT_SCALE * (ec * ec))
                corr = quad - base - cfm
                return acc + base * vm + cfm + sel * corr

            acc = plsc.parallel_loop(0, _A * NCH * _L, _L, unroll=2,
                                     carry=acc)(chunka)

        ub = wid * UW
        pb = P0 + ub
        psh = lax.rem(pb, 8)
        pltpu.sync_copy(sm_hbm.at[pl.ds(pl.multiple_of(pb - psh, 8), UWIN)], pu_v)
        gb = G0 + ub
        gal = pl.multiple_of(jnp.minimum(gb - lax.rem(gb, 8), STOT - UWIN), 8)
        gsh = gb - gal
        pltpu.sync_copy(sm_hbm.at[pl.ds(gal, UWIN)], gu_v)
        for c in range(NUF):
            dv = (gu_v[pl.ds(gsh + c * _L, _L)]
                  - pu_v[pl.ds(psh + c * _L, _L)])
            acc = acc + dv * dv
        tail = UW - NUF * _L
        if tail:
            um = jnp.where(lanev < _L - tail, _F32(0.0), _F32(1.0))
            dv = (gu_v[pl.ds(gsh + UW - _L, _L)]
                  - pu_v[pl.ds(psh + UW - _L, _L)]) * um
            acc = acc + dv * dv

        out_v[...] = acc * 0.5
        pltpu.sync_copy(out_v, out_hbm.at[wid])

    return sc_loss


def kernel(pred, pred_uvd, target, uvd_gt, train_out):
    B, H, W = pred.shape[0], pred.shape[2], pred.shape[3]
    U = pred_uvd.shape[1]
    sc_loss = _build_sc_call(B, H, W, U)
    ROW = _A * 5 * H * W
    ROWP = ((ROW + 15) // 16) * 16
    predf = jnp.pad(pred.reshape(B, ROW), ((0, 0), (0, ROWP - ROW)))
    smalls = jnp.concatenate(
        [target.reshape(-1), pred_uvd.reshape(-1), uvd_gt.reshape(-1)])
    partials = sc_loss(predf, smalls)
    return jnp.sum(partials)


# speedup vs baseline: 1.0770x; 1.0118x over previous
import functools

import jax
import jax.numpy as jnp
from jax import lax
from jax.experimental import pallas as pl
from jax.experimental.pallas import tpu as pltpu
from jax.experimental.pallas import tpu_sc as plsc

_ANCHORS = [1.3221, 1.73145, 3.19275, 4.00944, 5.05587, 8.09892,
            9.47112, 4.84053, 11.2364, 10.0071]
_A = 5
_OBJECT_SCALE = 5.0
_SIL_THRESH = 0.6
_L = 16

_F32 = jnp.float32
_I32 = jnp.int32


def _bcast_lane(v, i):
    idx = jnp.full((_L,), i, _I32)
    dnums = lax.GatherDimensionNumbers(
        offset_dims=(), collapsed_slice_dims=(0,), start_index_map=(0,))
    return lax.gather(v, idx[:, None], dnums, slice_sizes=(1,),
                      mode=lax.GatherScatterMode.PROMISE_IN_BOUNDS)


def _sig(x):
    return 1.0 / (1.0 + jnp.exp(-x))


def _vlog(x):
    bits = lax.bitcast_convert_type(x, _I32)
    e = (bits >> 23) - 127
    mbits = (bits & _I32(0x007FFFFF)) | _I32(0x3F800000)
    m = lax.bitcast_convert_type(mbits, _F32)
    big = m > 1.41421356237
    m = jnp.where(big, m * 0.5, m)
    e = e + jnp.where(big, 1, 0)
    t = m - 1.0
    z = t * t
    p = jnp.full((_L,), 7.0376836292e-2, _F32)
    for c in (-1.1514610310e-1, 1.1676998740e-1, -1.2420140846e-1,
              1.4249322787e-1, -1.6668057665e-1, 2.0000714765e-1,
              -2.4999993993e-1, 3.3333331174e-1):
        p = p * t + _F32(c)
    y = t * z * p - 0.5 * z
    return t + y + e.astype(_F32) * _F32(0.6931471805599453)


def _build_sc_call(B, H, W, U):
    HW = H * W
    ROW = _A * 5 * HW
    ROWP = ((ROW + 15) // 16) * 16
    nfull = HW // _L
    NCH = nfull + (1 if HW % _L else 0)
    GRID = NCH * _L
    try:
        info = plsc.get_sparse_core_info()
        NC, NS = info.num_cores, info.num_subcores
    except Exception:
        NC, NS = 2, 16
    NW = NC * NS
    BPW = B // NW
    UW = U * BPW
    UWIN = ((UW + 10 + 15) // 16) * 16
    UWIN = UW + 10
    NUF = UW // _L
    T0 = 0
    P0 = 4 * B
    G0 = P0 + U * B
    STOT = G0 + U * B

    mesh = plsc.VectorSubcoreMesh(core_axis_name="c", subcore_axis_name="s")

    @functools.partial(
        pl.kernel, mesh=mesh,
        out_type=jax.ShapeDtypeStruct((NW, _L), _F32),
        scratch_types=[
            pltpu.VMEM((ROW,), _F32),
            pltpu.VMEM((_L,), _F32),
            pltpu.VMEM((UWIN,), _F32),
            pltpu.VMEM((UWIN,), _F32),
            pltpu.VMEM((_L,), _F32),
            pltpu.VMEM((GRID,), _F32),
            pltpu.VMEM((GRID,), _F32),
            pltpu.VMEM((GRID,), _I32),
            pltpu.VMEM((GRID,), _F32),
        ],
    )
    def sc_loss(pred_hbm, sm_hbm, out_hbm,
                pred_v, targ_v, pu_v, gu_v, out_v,
                wg_v, hg_v, psc_v, vm_v):
        wid = lax.axis_index("s") * NC + lax.axis_index("c")
        zero = jnp.zeros((_L,), _F32)
        acc = zero

        lanev = lax.iota(_I32, _L)
        ones = jnp.full((_L,), 1.0, _F32)
        for j in range(NCH):
            off = min(j * _L, HW - _L)
            pos = lanev + off
            ndup = j * _L - off
            if j * _L + _L > HW:
                psc_v[pl.ds(j * _L, _L)] = jnp.where(lanev < ndup, -1, pos)
                vm_v[pl.ds(j * _L, _L)] = jnp.where(lanev < ndup, _F32(0.0),
                                                    _F32(1.0))
            else:
                psc_v[pl.ds(j * _L, _L)] = pos
                vm_v[pl.ds(j * _L, _L)] = ones
            wg_v[pl.ds(j * _L, _L)] = lax.rem(pos, W).astype(_F32)
            hg_v[pl.ds(j * _L, _L)] = lax.div(pos, W).astype(_F32)

        for k in range(BPW):
            b = wid * BPW + k
            pltpu.sync_copy(pred_hbm.at[b], pred_v)
            tb = b * 4
            tsh = lax.rem(tb, 8)
            pltpu.sync_copy(sm_hbm.at[pl.ds(pl.multiple_of(tb - tsh, 8), _L)], targ_v)
            tv = targ_v[...]
            gxv = _bcast_lane(tv, tsh) * _F32(W)
            gyv = _bcast_lane(tv, tsh + 1) * _F32(H)
            gwv = _bcast_lane(tv, tsh + 2) * _F32(W)
            ghv = _bcast_lane(tv, tsh + 3) * _F32(H)
            gxl = gxv - gwv * 0.5
            gxr = gxv + gwv * 0.5
            gyl = gyv - ghv * 0.5
            gyr = gyv + ghv * 0.5
            garea = gwv * ghv

            bestv = jnp.zeros((_L,), _I32)
            biou = None
            for a in range(_A):
                awa = _ANCHORS[2 * a]
                aha = _ANCHORS[2 * a + 1]
                uw = jnp.maximum(gwv, _F32(awa))
                uh = jnp.maximum(ghv, _F32(aha))
                cw = (gwv + _F32(awa)) - uw
                ch = (ghv + _F32(aha)) - uh
                carea = jnp.maximum(cw, 0.0) * jnp.maximum(ch, 0.0)
                uarea = (_F32(awa * aha) + garea) - carea
                au = carea / uarea
                if biou is None:
                    biou = au
                else:
                    upd = au > biou
                    bestv = jnp.where(upd, a, bestv)
                    biou = jnp.where(upd, au, biou)
            awbv = zero
            ahbv = zero
            for a in range(_A):
                hit = bestv == a
                awbv = awbv + jnp.where(hit, _F32(_ANCHORS[2 * a]), 0.0)
                ahbv = ahbv + jnp.where(hit, _F32(_ANCHORS[2 * a + 1]), 0.0)
            lwv = _vlog(gwv / awbv)
            lhv = _vlog(ghv / ahbv)
            giv = gxv.astype(_I32)
            gjv = gyv.astype(_I32)
            dxv = gxv - giv.astype(_F32)
            dyv = gyv - gjv.astype(_F32)
            pselv = gjv * W + giv

            def chunka(t, acc, gxl=gxl, gxr=gxr, gyl=gyl, gyr=gyr,
                       garea=garea, gwv=gwv, ghv=ghv, dxv=dxv, dyv=dyv,
                       lwv=lwv, lhv=lhv, pselv=pselv, bestv=bestv):
                a = lax.div(t, NCH * _L)
                go = t - a * (NCH * _L)
                awa = jnp.full((_L,), _ANCHORS[2 * (_A - 1)], _F32)
                aha = jnp.full((_L,), _ANCHORS[2 * (_A - 1) + 1], _F32)
                for aa in range(_A - 1):
                    awa = jnp.where(a == aa, _F32(_ANCHORS[2 * aa]), awa)
                    aha = jnp.where(a == aa, _F32(_ANCHORS[2 * aa + 1]), aha)
                bm = jnp.where(bestv == a, _F32(1.0), _F32(0.0))
                off = jnp.minimum(go, HW - _L) + a * (5 * HW)
                xr = pred_v[pl.ds(off, _L)]
                yr = pred_v[pl.ds(off + HW, _L)]
                twv = pred_v[pl.ds(off + 2 * HW, _L)]
                thv = pred_v[pl.ds(off + 3 * HW, _L)]
                cr = pred_v[pl.ds(off + 4 * HW, _L)]
                wg = wg_v[pl.ds(go, _L)]
                hg = hg_v[pl.ds(go, _L)]
                psc = psc_v[pl.ds(go, _L)]
                vm = vm_v[pl.ds(go, _L)]
                ax = 1.0 + jnp.exp(-xr)
                ay = 1.0 + jnp.exp(-yr)
                acf = 1.0 + jnp.exp(-cr)
                axy = ax * ay
                bwv = jnp.exp(twv) * awa
                bhv = jnp.exp(thv) * aha
                rx = 1.0 / axy
                sx = rx * ay
                sy = rx * ax
                bxv = sx + wg
                byv = sy + hg
                mx = jnp.minimum(bxv - bwv * 0.5, gxl)
                nx = jnp.maximum(bxv + bwv * 0.5, gxr)
                my = jnp.minimum(byv - bhv * 0.5, gyl)
                ny = jnp.maximum(byv + bhv * 0.5, gyr)
                cw = (bwv + gwv) - (nx - mx)
                ch = (bhv + ghv) - (ny - my)
                carea = jnp.maximum(cw, 0.0) * jnp.maximum(ch, 0.0)
                uarea = (bwv * bhv + garea) - carea
                den = acf * uarea
                rcu = 1.0 / den
                cf = rcu * uarea
                iou = (rcu * acf) * carea
                m01 = jnp.where(iou > _SIL_THRESH, _F32(0.0), vm)
                sxc = sx - 0.5
                syc = sy - 0.5
                base = sxc * sxc + syc * syc + twv * twv + thv * thv
                cfm = cf * cf * m01
                sel = jnp.where(psc == pselv, bm, _F32(0.0))
                ex = sx - dxv
                ey = sy - dyv
                ew = twv - lwv
                eh = thv - lhv
                ec = cf - iou
                quad = (ex * ex + ey * ey + ew * ew + eh * eh
                        + _OBJECT_SCALE * (ec * ec))
                corr = quad - base - cfm
                return acc + base * vm + cfm + sel * corr

            acc = plsc.parallel_loop(0, _A * NCH * _L, _L, unroll=2,
                                     carry=acc)(chunka)

        ub = wid * UW
        pb = P0 + ub
        psh = lax.rem(pb, 8)
        pltpu.sync_copy(sm_hbm.at[pl.ds(pl.multiple_of(pb - psh, 8), UWIN)], pu_v)
        gb = G0 + ub
        gal = pl.multiple_of(jnp.minimum(gb - lax.rem(gb, 8), STOT - UWIN), 8)
        gsh = gb - gal
        pltpu.sync_copy(sm_hbm.at[pl.ds(gal, UWIN)], gu_v)
        for c in range(NUF):
            dv = (gu_v[pl.ds(gsh + c * _L, _L)]
                  - pu_v[pl.ds(psh + c * _L, _L)])
            acc = acc + dv * dv
        tail = UW - NUF * _L
        if tail:
            um = jnp.where(lanev < _L - tail, _F32(0.0), _F32(1.0))
            dv = (gu_v[pl.ds(gsh + UW - _L, _L)]
                  - pu_v[pl.ds(psh + UW - _L, _L)]) * um
            acc = acc + dv * dv

        out_v[...] = acc * 0.5
        pltpu.sync_copy(out_v, out_hbm.at[wid])

    return sc_loss


def kernel(pred, pred_uvd, target, uvd_gt, train_out):
    B, H, W = pred.shape[0], pred.shape[2], pred.shape[3]
    U = pred_uvd.shape[1]
    sc_loss = _build_sc_call(B, H, W, U)
    ROW = _A * 5 * H * W
    ROWP = ((ROW + 15) // 16) * 16
    predf = pred.reshape(B, ROW)
    smalls = jnp.concatenate(
        [target.reshape(-1), pred_uvd.reshape(-1), uvd_gt.reshape(-1)])
    partials = sc_loss(predf, smalls)
    return jnp.sum(partials)


# speedup vs baseline: 1.1728x; 1.0889x over previous
import functools

import jax
import jax.numpy as jnp
from jax import lax
from jax.experimental import pallas as pl
from jax.experimental.pallas import tpu as pltpu
from jax.experimental.pallas import tpu_sc as plsc

_ANCHORS = [1.3221, 1.73145, 3.19275, 4.00944, 5.05587, 8.09892,
            9.47112, 4.84053, 11.2364, 10.0071]
_A = 5
_OBJECT_SCALE = 5.0
_SIL_THRESH = 0.6
_L = 16

_F32 = jnp.float32
_I32 = jnp.int32


def _bcast_lane(v, i):
    idx = jnp.full((_L,), i, _I32)
    dnums = lax.GatherDimensionNumbers(
        offset_dims=(), collapsed_slice_dims=(0,), start_index_map=(0,))
    return lax.gather(v, idx[:, None], dnums, slice_sizes=(1,),
                      mode=lax.GatherScatterMode.PROMISE_IN_BOUNDS)


def _sig(x):
    return 1.0 / (1.0 + jnp.exp(-x))


def _vlog(x):
    bits = lax.bitcast_convert_type(x, _I32)
    e = (bits >> 23) - 127
    mbits = (bits & _I32(0x007FFFFF)) | _I32(0x3F800000)
    m = lax.bitcast_convert_type(mbits, _F32)
    big = m > 1.41421356237
    m = jnp.where(big, m * 0.5, m)
    e = e + jnp.where(big, 1, 0)
    t = m - 1.0
    z = t * t
    p = jnp.full((_L,), 7.0376836292e-2, _F32)
    for c in (-1.1514610310e-1, 1.1676998740e-1, -1.2420140846e-1,
              1.4249322787e-1, -1.6668057665e-1, 2.0000714765e-1,
              -2.4999993993e-1, 3.3333331174e-1):
        p = p * t + _F32(c)
    y = t * z * p - 0.5 * z
    return t + y + e.astype(_F32) * _F32(0.6931471805599453)


def _build_sc_call(B, H, W, U):
    HW = H * W
    ROW = _A * 5 * HW
    ROWP = ((ROW + 15) // 16) * 16
    nfull = HW // _L
    NCH = nfull + (1 if HW % _L else 0)
    GRID = NCH * _L
    try:
        info = plsc.get_sparse_core_info()
        NC, NS = info.num_cores, info.num_subcores
    except Exception:
        NC, NS = 2, 16
    NW = NC * NS
    BPW = B // NW
    UW = U * BPW
    UWIN = ((UW + 10 + 15) // 16) * 16
    UWIN = UW + 10
    NUF = UW // _L
    T0 = 0
    P0 = 4 * B
    G0 = P0 + U * B
    STOT = G0 + U * B

    mesh = plsc.VectorSubcoreMesh(core_axis_name="c", subcore_axis_name="s")

    @functools.partial(
        pl.kernel, mesh=mesh,
        out_type=jax.ShapeDtypeStruct((NW, _L), _F32),
        scratch_types=[
            pltpu.VMEM((ROW,), _F32),
            pltpu.VMEM((ROW,), _F32),
            pltpu.VMEM((_L,), _F32),
            pltpu.VMEM((_L,), _F32),
            pltpu.VMEM((UWIN,), _F32),
            pltpu.VMEM((UWIN,), _F32),
            pltpu.VMEM((_L,), _F32),
            pltpu.VMEM((GRID,), _F32),
            pltpu.VMEM((GRID,), _F32),
            pltpu.VMEM((GRID,), _I32),
            pltpu.VMEM((GRID,), _F32),
        ] + [pltpu.SemaphoreType.DMA] * 6,
    )
    def sc_loss(pred_hbm, sm_hbm, out_hbm,
                pred_va, pred_vb, targ_va, targ_vb, pu_v, gu_v, out_v,
                wg_v, hg_v, psc_v, vm_v,
                sp0, sp1, st0, st1, su0, su1):
        wid = lax.axis_index("s") * NC + lax.axis_index("c")
        zero = jnp.zeros((_L,), _F32)
        acc = zero

        lanev = lax.iota(_I32, _L)
        ones = jnp.full((_L,), 1.0, _F32)
        for j in range(NCH):
            off = min(j * _L, HW - _L)
            pos = lanev + off
            ndup = j * _L - off
            if j * _L + _L > HW:
                psc_v[pl.ds(j * _L, _L)] = jnp.where(lanev < ndup, -1, pos)
                vm_v[pl.ds(j * _L, _L)] = jnp.where(lanev < ndup, _F32(0.0),
                                                    _F32(1.0))
            else:
                psc_v[pl.ds(j * _L, _L)] = pos
                vm_v[pl.ds(j * _L, _L)] = ones
            wg_v[pl.ds(j * _L, _L)] = lax.rem(pos, W).astype(_F32)
            hg_v[pl.ds(j * _L, _L)] = lax.div(pos, W).astype(_F32)

        hp, ht, tshs = [], [], []
        for k in range(BPW):
            b = wid * BPW + k
            tb = b * 4
            tsh = lax.rem(tb, 8)
            tshs.append(tsh)
            hp.append(pltpu.async_copy(pred_hbm.at[b],
                                       (pred_va, pred_vb)[k],
                                       (sp0, sp1)[k]))
            ht.append(pltpu.async_copy(
                sm_hbm.at[pl.ds(pl.multiple_of(tb - tsh, 8), _L)],
                (targ_va, targ_vb)[k], (st0, st1)[k]))
        ub = wid * UW
        pb = P0 + ub
        psh = lax.rem(pb, 8)
        hu0 = pltpu.async_copy(sm_hbm.at[pl.ds(pl.multiple_of(pb - psh, 8), UWIN)], pu_v, su0)
        gb = G0 + ub
        gal = pl.multiple_of(jnp.minimum(gb - lax.rem(gb, 8), STOT - UWIN), 8)
        gsh = gb - gal
        hu1 = pltpu.async_copy(sm_hbm.at[pl.ds(gal, UWIN)], gu_v, su1)

        for k in range(BPW):
            b = wid * BPW + k
            tsh = tshs[k]
            ht[k].wait()
            tv = (targ_va, targ_vb)[k][...]
            gxv = _bcast_lane(tv, tsh) * _F32(W)
            gyv = _bcast_lane(tv, tsh + 1) * _F32(H)
            gwv = _bcast_lane(tv, tsh + 2) * _F32(W)
            ghv = _bcast_lane(tv, tsh + 3) * _F32(H)
            gxl = gxv - gwv * 0.5
            gxr = gxv + gwv * 0.5
            gyl = gyv - ghv * 0.5
            gyr = gyv + ghv * 0.5
            garea = gwv * ghv

            bestv = jnp.zeros((_L,), _I32)
            biou = None
            for a in range(_A):
                awa = _ANCHORS[2 * a]
                aha = _ANCHORS[2 * a + 1]
                uw = jnp.maximum(gwv, _F32(awa))
                uh = jnp.maximum(ghv, _F32(aha))
                cw = (gwv + _F32(awa)) - uw
                ch = (ghv + _F32(aha)) - uh
                carea = jnp.maximum(cw, 0.0) * jnp.maximum(ch, 0.0)
                uarea = (_F32(awa * aha) + garea) - carea
                au = carea / uarea
                if biou is None:
                    biou = au
                else:
                    upd = au > biou
                    bestv = jnp.where(upd, a, bestv)
                    biou = jnp.where(upd, au, biou)
            awbv = zero
            ahbv = zero
            for a in range(_A):
                hit = bestv == a
                awbv = awbv + jnp.where(hit, _F32(_ANCHORS[2 * a]), 0.0)
                ahbv = ahbv + jnp.where(hit, _F32(_ANCHORS[2 * a + 1]), 0.0)
            lwv = _vlog(gwv / awbv)
            lhv = _vlog(ghv / ahbv)
            giv = gxv.astype(_I32)
            gjv = gyv.astype(_I32)
            dxv = gxv - giv.astype(_F32)
            dyv = gyv - gjv.astype(_F32)
            pselv = gjv * W + giv
            hp[k].wait()
            pred_v = (pred_va, pred_vb)[k]

            def chunka(t, acc, gxl=gxl, gxr=gxr, gyl=gyl, gyr=gyr,
                       garea=garea, gwv=gwv, ghv=ghv, dxv=dxv, dyv=dyv,
                       lwv=lwv, lhv=lhv, pselv=pselv, bestv=bestv):
                a = lax.div(t, NCH * _L)
                go = t - a * (NCH * _L)
                awa = jnp.full((_L,), _ANCHORS[2 * (_A - 1)], _F32)
                aha = jnp.full((_L,), _ANCHORS[2 * (_A - 1) + 1], _F32)
                for aa in range(_A - 1):
                    awa = jnp.where(a == aa, _F32(_ANCHORS[2 * aa]), awa)
                    aha = jnp.where(a == aa, _F32(_ANCHORS[2 * aa + 1]), aha)
                bm = jnp.where(bestv == a, _F32(1.0), _F32(0.0))
                off = jnp.minimum(go, HW - _L) + a * (5 * HW)
                xr = pred_v[pl.ds(off, _L)]
                yr = pred_v[pl.ds(off + HW, _L)]
                twv = pred_v[pl.ds(off + 2 * HW, _L)]
                thv = pred_v[pl.ds(off + 3 * HW, _L)]
                cr = pred_v[pl.ds(off + 4 * HW, _L)]
                wg = wg_v[pl.ds(go, _L)]
                hg = hg_v[pl.ds(go, _L)]
                psc = psc_v[pl.ds(go, _L)]
                vm = vm_v[pl.ds(go, _L)]
                ax = 1.0 + jnp.exp(-xr)
                ay = 1.0 + jnp.exp(-yr)
                acf = 1.0 + jnp.exp(-cr)
                axy = ax * ay
                bwv = jnp.exp(twv) * awa
                bhv = jnp.exp(thv) * aha
                rx = 1.0 / axy
                sx = rx * ay
                sy = rx * ax
                bxv = sx + wg
                byv = sy + hg
                mx = jnp.minimum(bxv - bwv * 0.5, gxl)
                nx = jnp.maximum(bxv + bwv * 0.5, gxr)
                my = jnp.minimum(byv - bhv * 0.5, gyl)
                ny = jnp.maximum(byv + bhv * 0.5, gyr)
                cw = (bwv + gwv) - (nx - mx)
                ch = (bhv + ghv) - (ny - my)
                carea = jnp.maximum(cw, 0.0) * jnp.maximum(ch, 0.0)
                uarea = (bwv * bhv + garea) - carea
                den = acf * uarea
                rcu = 1.0 / den
                cf = rcu * uarea
                iou = (rcu * acf) * carea
                m01 = jnp.where(iou > _SIL_THRESH, _F32(0.0), vm)
                sxc = sx - 0.5
                syc = sy - 0.5
                base = sxc * sxc + syc * syc + twv * twv + thv * thv
                cfm = cf * cf * m01
                sel = jnp.where(psc == pselv, bm, _F32(0.0))
                ex = sx - dxv
                ey = sy - dyv
                ew = twv - lwv
                eh = thv - lhv
                ec = cf - iou
                quad = (ex * ex + ey * ey + ew * ew + eh * eh
                        + _OBJECT_SCALE * (ec * ec))
                corr = quad - base - cfm
                return acc + base * vm + cfm + sel * corr

            acc = plsc.parallel_loop(0, _A * NCH * _L, _L, unroll=2,
                                     carry=acc)(chunka)

        hu0.wait()
        hu1.wait()
        for c in range(NUF):
            dv = (gu_v[pl.ds(gsh + c * _L, _L)]
                  - pu_v[pl.ds(psh + c * _L, _L)])
            acc = acc + dv * dv
        tail = UW - NUF * _L
        if tail:
            um = jnp.where(lanev < _L - tail, _F32(0.0), _F32(1.0))
            dv = (gu_v[pl.ds(gsh + UW - _L, _L)]
                  - pu_v[pl.ds(psh + UW - _L, _L)]) * um
            acc = acc + dv * dv

        out_v[...] = acc * 0.5
        pltpu.sync_copy(out_v, out_hbm.at[wid])

    return sc_loss


def kernel(pred, pred_uvd, target, uvd_gt, train_out):
    B, H, W = pred.shape[0], pred.shape[2], pred.shape[3]
    U = pred_uvd.shape[1]
    sc_loss = _build_sc_call(B, H, W, U)
    ROW = _A * 5 * H * W
    ROWP = ((ROW + 15) // 16) * 16
    predf = pred.reshape(B, ROW)
    smalls = jnp.concatenate(
        [target.reshape(-1), pred_uvd.reshape(-1), uvd_gt.reshape(-1)])
    partials = sc_loss(predf, smalls)
    return jnp.sum(partials)


# speedup vs baseline: 1.1908x; 1.0154x over previous
import functools

import jax
import jax.numpy as jnp
from jax import lax
from jax.experimental import pallas as pl
from jax.experimental.pallas import tpu as pltpu
from jax.experimental.pallas import tpu_sc as plsc

_ANCHORS = [1.3221, 1.73145, 3.19275, 4.00944, 5.05587, 8.09892,
            9.47112, 4.84053, 11.2364, 10.0071]
_A = 5
_OBJECT_SCALE = 5.0
_SIL_THRESH = 0.6
_L = 16

_F32 = jnp.float32
_I32 = jnp.int32


def _bcast_lane(v, i):
    idx = jnp.full((_L,), i, _I32)
    dnums = lax.GatherDimensionNumbers(
        offset_dims=(), collapsed_slice_dims=(0,), start_index_map=(0,))
    return lax.gather(v, idx[:, None], dnums, slice_sizes=(1,),
                      mode=lax.GatherScatterMode.PROMISE_IN_BOUNDS)


def _sig(x):
    return 1.0 / (1.0 + jnp.exp(-x))


def _vlog(x):
    bits = lax.bitcast_convert_type(x, _I32)
    e = (bits >> 23) - 127
    mbits = (bits & _I32(0x007FFFFF)) | _I32(0x3F800000)
    m = lax.bitcast_convert_type(mbits, _F32)
    big = m > 1.41421356237
    m = jnp.where(big, m * 0.5, m)
    e = e + jnp.where(big, 1, 0)
    t = m - 1.0
    z = t * t
    p = jnp.full((_L,), 7.0376836292e-2, _F32)
    for c in (-1.1514610310e-1, 1.1676998740e-1, -1.2420140846e-1,
              1.4249322787e-1, -1.6668057665e-1, 2.0000714765e-1,
              -2.4999993993e-1, 3.3333331174e-1):
        p = p * t + _F32(c)
    y = t * z * p - 0.5 * z
    return t + y + e.astype(_F32) * _F32(0.6931471805599453)


def _build_sc_call(B, H, W, U):
    HW = H * W
    ROW = _A * 5 * HW
    ROWP = ((ROW + 15) // 16) * 16
    nfull = HW // _L
    NCH = nfull + (1 if HW % _L else 0)
    GRID = NCH * _L
    try:
        info = plsc.get_sparse_core_info()
        NC, NS = info.num_cores, info.num_subcores
    except Exception:
        NC, NS = 2, 16
    NW = NC * NS
    BPW = B // NW
    UW = U * BPW
    UWIN = ((UW + 10 + 15) // 16) * 16
    UWIN = UW + 10
    NUF = UW // _L
    T0 = 0
    P0 = 4 * B
    G0 = P0 + U * B
    STOT = G0 + U * B

    mesh = plsc.VectorSubcoreMesh(core_axis_name="c", subcore_axis_name="s")

    @functools.partial(
        pl.kernel, mesh=mesh,
        out_type=jax.ShapeDtypeStruct((NW, _L), _F32),
        scratch_types=[
            pltpu.VMEM((ROW,), _F32),
            pltpu.VMEM((ROW,), _F32),
            pltpu.VMEM((_L,), _F32),
            pltpu.VMEM((_L,), _F32),
            pltpu.VMEM((U,), _F32),
            pltpu.VMEM((U,), _F32),
            pltpu.VMEM((U,), _F32),
            pltpu.VMEM((U,), _F32),
            pltpu.VMEM((_L,), _F32),
            pltpu.VMEM((GRID,), _F32),
            pltpu.VMEM((GRID,), _F32),
            pltpu.VMEM((GRID,), _I32),
            pltpu.VMEM((GRID,), _F32),
        ] + [pltpu.SemaphoreType.DMA] * 8,
    )
    def sc_loss(pred_hbm, tg_hbm, pu_hbm, gu_hbm, out_hbm,
                pred_va, pred_vb, targ_va, targ_vb,
                pu_va, pu_vb, gu_va, gu_vb, out_v,
                wg_v, hg_v, psc_v, vm_v,
                sp0, sp1, st0, st1, su0, su1, su2, su3):
        wid = lax.axis_index("s") * NC + lax.axis_index("c")
        zero = jnp.zeros((_L,), _F32)
        acc = zero

        lanev = lax.iota(_I32, _L)
        ones = jnp.full((_L,), 1.0, _F32)
        for j in range(NCH):
            off = min(j * _L, HW - _L)
            pos = lanev + off
            ndup = j * _L - off
            if j * _L + _L > HW:
                psc_v[pl.ds(j * _L, _L)] = jnp.where(lanev < ndup, -1, pos)
                vm_v[pl.ds(j * _L, _L)] = jnp.where(lanev < ndup, _F32(0.0),
                                                    _F32(1.0))
            else:
                psc_v[pl.ds(j * _L, _L)] = pos
                vm_v[pl.ds(j * _L, _L)] = ones
            wg_v[pl.ds(j * _L, _L)] = lax.rem(pos, W).astype(_F32)
            hg_v[pl.ds(j * _L, _L)] = lax.div(pos, W).astype(_F32)

        hp, ht, tshs = [], [], []
        for k in range(BPW):
            b = wid * BPW + k
            tb = b * 4
            tal = jnp.minimum(tb - lax.rem(tb, 8), 4 * B - _L)
            tsh = tb - tal
            tshs.append(tsh)
            hp.append(pltpu.async_copy(pred_hbm.at[b],
                                       (pred_va, pred_vb)[k],
                                       (sp0, sp1)[k]))
            ht.append(pltpu.async_copy(
                tg_hbm.at[pl.ds(pl.multiple_of(tal, 8), _L)],
                (targ_va, targ_vb)[k], (st0, st1)[k]))
        hu = []
        for k in range(BPW):
            b = wid * BPW + k
            hu.append(pltpu.async_copy(pu_hbm.at[b], (pu_va, pu_vb)[k],
                                       (su0, su1)[k]))
            hu.append(pltpu.async_copy(gu_hbm.at[b], (gu_va, gu_vb)[k],
                                       (su2, su3)[k]))

        for k in range(BPW):
            b = wid * BPW + k
            tsh = tshs[k]
            ht[k].wait()
            tv = (targ_va, targ_vb)[k][...]
            gxv = _bcast_lane(tv, tsh) * _F32(W)
            gyv = _bcast_lane(tv, tsh + 1) * _F32(H)
            gwv = _bcast_lane(tv, tsh + 2) * _F32(W)
            ghv = _bcast_lane(tv, tsh + 3) * _F32(H)
            gxl = gxv - gwv * 0.5
            gxr = gxv + gwv * 0.5
            gyl = gyv - ghv * 0.5
            gyr = gyv + ghv * 0.5
            garea = gwv * ghv

            bestv = jnp.zeros((_L,), _I32)
            biou = None
            for a in range(_A):
                awa = _ANCHORS[2 * a]
                aha = _ANCHORS[2 * a + 1]
                uw = jnp.maximum(gwv, _F32(awa))
                uh = jnp.maximum(ghv, _F32(aha))
                cw = (gwv + _F32(awa)) - uw
                ch = (ghv + _F32(aha)) - uh
                carea = jnp.maximum(cw, 0.0) * jnp.maximum(ch, 0.0)
                uarea = (_F32(awa * aha) + garea) - carea
                au = carea / uarea
                if biou is None:
                    biou = au
                else:
                    upd = au > biou
                    bestv = jnp.where(upd, a, bestv)
                    biou = jnp.where(upd, au, biou)
            awbv = zero
            ahbv = zero
            for a in range(_A):
                hit = bestv == a
                awbv = awbv + jnp.where(hit, _F32(_ANCHORS[2 * a]), 0.0)
                ahbv = ahbv + jnp.where(hit, _F32(_ANCHORS[2 * a + 1]), 0.0)
            lwv = _vlog(gwv / awbv)
            lhv = _vlog(ghv / ahbv)
            giv = gxv.astype(_I32)
            gjv = gyv.astype(_I32)
            dxv = gxv - giv.astype(_F32)
            dyv = gyv - gjv.astype(_F32)
            pselv = gjv * W + giv
            hp[k].wait()
            pred_v = (pred_va, pred_vb)[k]

            def chunka(t, acc, gxl=gxl, gxr=gxr, gyl=gyl, gyr=gyr,
                       garea=garea, gwv=gwv, ghv=ghv, dxv=dxv, dyv=dyv,
                       lwv=lwv, lhv=lhv, pselv=pselv, bestv=bestv):
                a = lax.div(t, NCH * _L)
                go = t - a * (NCH * _L)
                awa = jnp.full((_L,), _ANCHORS[2 * (_A - 1)], _F32)
                aha = jnp.full((_L,), _ANCHORS[2 * (_A - 1) + 1], _F32)
                for aa in range(_A - 1):
                    awa = jnp.where(a == aa, _F32(_ANCHORS[2 * aa]), awa)
                    aha = jnp.where(a == aa, _F32(_ANCHORS[2 * aa + 1]), aha)
                bm = jnp.where(bestv == a, _F32(1.0), _F32(0.0))
                off = jnp.minimum(go, HW - _L) + a * (5 * HW)
                xr = pred_v[pl.ds(off, _L)]
                yr = pred_v[pl.ds(off + HW, _L)]
                twv = pred_v[pl.ds(off + 2 * HW, _L)]
                thv = pred_v[pl.ds(off + 3 * HW, _L)]
                cr = pred_v[pl.ds(off + 4 * HW, _L)]
                wg = wg_v[pl.ds(go, _L)]
                hg = hg_v[pl.ds(go, _L)]
                psc = psc_v[pl.ds(go, _L)]
                vm = vm_v[pl.ds(go, _L)]
                ax = 1.0 + jnp.exp(-xr)
                ay = 1.0 + jnp.exp(-yr)
                acf = 1.0 + jnp.exp(-cr)
                axy = ax * ay
                bwv = jnp.exp(twv) * awa
                bhv = jnp.exp(thv) * aha
                rx = 1.0 / axy
                sx = rx * ay
                sy = rx * ax
                bxv = sx + wg
                byv = sy + hg
                mx = jnp.minimum(bxv - bwv * 0.5, gxl)
                nx = jnp.maximum(bxv + bwv * 0.5, gxr)
                my = jnp.minimum(byv - bhv * 0.5, gyl)
                ny = jnp.maximum(byv + bhv * 0.5, gyr)
                cw = (bwv + gwv) - (nx - mx)
                ch = (bhv + ghv) - (ny - my)
                carea = jnp.maximum(cw, 0.0) * jnp.maximum(ch, 0.0)
                uarea = (bwv * bhv + garea) - carea
                den = acf * uarea
                rcu = 1.0 / den
                cf = rcu * uarea
                iou = (rcu * acf) * carea
                m01 = jnp.where(iou > _SIL_THRESH, _F32(0.0), vm)
                sxc = sx - 0.5
                syc = sy - 0.5
                base = sxc * sxc + syc * syc + twv * twv + thv * thv
                cfm = cf * cf * m01
                sel = jnp.where(psc == pselv, bm, _F32(0.0))
                ex = sx - dxv
                ey = sy - dyv
                ew = twv - lwv
                eh = thv - lhv
                ec = cf - iou
                quad = (ex * ex + ey * ey + ew * ew + eh * eh
                        + _OBJECT_SCALE * (ec * ec))
                corr = quad - base - cfm
                return acc + base * vm + cfm + sel * corr

            acc = plsc.parallel_loop(0, _A * NCH * _L, _L, unroll=2,
                                     carry=acc)(chunka)

        for h in hu:
            h.wait()
        nuf = U // _L
        tail = U - nuf * _L
        um = jnp.where(lanev < _L - tail, _F32(0.0), _F32(1.0))
        for k in range(BPW):
            puv = (pu_va, pu_vb)[k]
            guv = (gu_va, gu_vb)[k]
            for c in range(nuf):
                dv = guv[pl.ds(c * _L, _L)] - puv[pl.ds(c * _L, _L)]
                acc = acc + dv * dv
            if tail:
                dv = (guv[pl.ds(U - _L, _L)] - puv[pl.ds(U - _L, _L)]) * um
                acc = acc + dv * dv

        out_v[...] = acc * 0.5
        pltpu.sync_copy(out_v, out_hbm.at[wid])

    return sc_loss


def kernel(pred, pred_uvd, target, uvd_gt, train_out):
    B, H, W = pred.shape[0], pred.shape[2], pred.shape[3]
    U = pred_uvd.shape[1]
    sc_loss = _build_sc_call(B, H, W, U)
    ROW = _A * 5 * H * W
    ROWP = ((ROW + 15) // 16) * 16
    predf = pred.reshape(B, ROW)
    partials = sc_loss(predf, target.reshape(-1), pred_uvd, uvd_gt)
    return jnp.sum(partials)
